# baseline (speedup 1.0000x reference)
import functools

import jax
import jax.numpy as jnp
from jax import lax
from jax.experimental import pallas as pl
from jax.experimental.pallas import tpu as pltpu
from jax.experimental.pallas import tpu_sc as plsc

NUM_B = 4096
NUM_T = 200
NUM_TOKENS = NUM_B * NUM_T
DIM = 64
NUM_WORKERS = 32
PER_WORKER = NUM_TOKENS // NUM_WORKERS
CHUNK = 128
NUM_CHUNKS = PER_WORKER // CHUNK
CHUNKS_PER_T = NUM_B // CHUNK
NBUF = 2
SUPER = NUM_CHUNKS // (2 * NBUF)
LANES = 16


def _embedding_gather_call():
    mesh = plsc.VectorSubcoreMesh(core_axis_name="c", subcore_axis_name="s")

    @functools.partial(
        pl.kernel,
        mesh=mesh,
        out_type=jax.ShapeDtypeStruct((NUM_T * DIM, NUM_B), jnp.float32),
        compiler_params=pltpu.CompilerParams(
            use_tc_tiling_on_sc=False, needs_layout_passes=False),
        scratch_types=(
            [pltpu.VMEM((NUM_CHUNKS, CHUNK), jnp.int32)]
            + [pltpu.VMEM((CHUNK, DIM), jnp.float32) for _ in range(2 * NBUF)]
            + [pltpu.VMEM((DIM, CHUNK), jnp.float32) for _ in range(2 * NBUF)]
            + [pltpu.SemaphoreType.DMA for _ in range(4)]
        ),
    )
    def gather_kernel(idx_hbm, table_hbm, out_hbm, idx_all, *bufs_and_sems):
        rows = bufs_and_sems[: 2 * NBUF]
        trans = bufs_and_sems[2 * NBUF: 4 * NBUF]
        gsem_a, ssem_a, gsem_b, ssem_b = bufs_and_sems[4 * NBUF:]
        rows_a, rows_b = rows[:NBUF], rows[NBUF:]
        trans_a, trans_b = trans[:NBUF], trans[NBUF:]

        wid = lax.axis_index("s") * 2 + lax.axis_index("c")
        crow = wid * NUM_CHUNKS

        pltpu.sync_copy(idx_hbm.at[pl.ds(crow, NUM_CHUNKS)], idx_all)

        def dst_slice(c):
            trow = (c // CHUNKS_PER_T) * DIM
            b0 = (c % CHUNKS_PER_T) * CHUNK
            return out_hbm.at[pl.ds(trow, DIM), pl.ds(b0, CHUNK)]

        def start_gathers(ci, bank_rows, gsem):
            return [
                pltpu.async_copy(
                    table_hbm.at[idx_all.at[ci + b]], bank_rows[b], gsem)
                for b in range(NBUF)
            ]

        def transpose_chunk(src, dst):
            def dloop(d, carry):
                cols = jnp.full((LANES,), d, jnp.int32)
                for g in range(CHUNK // LANES):
                    rw = lax.iota(jnp.int32, LANES) + g * LANES
                    dst[d, pl.ds(g * LANES, LANES)] = (
                        plsc.load_gather(src, [rw, cols]))
                return carry
            lax.fori_loop(0, DIM, dloop, 0)

        def process_bank(ci, bank_rows, bank_trans, ssem):
            for b in range(NBUF):
                transpose_chunk(bank_rows[b], bank_trans[b])
                pltpu.async_copy(
                    bank_trans[b], dst_slice(crow + ci + b), ssem)

        def drain_stores(bank_trans, ssem):
            for b in range(NBUF):
                pltpu.make_async_copy(
                    bank_trans[b],
                    out_hbm.at[pl.ds(0, DIM), pl.ds(0, CHUNK)], ssem).wait()

        def body(s, carry):
            ci_a = s * 2 * NBUF
            ci_b = ci_a + NBUF

            @pl.when(s > 0)
            def _():
                drain_stores(trans_a, ssem_a)

            ga = start_gathers(ci_a, rows_a, gsem_a)

            @pl.when(s > 0)
            def _():
                drain_stores(trans_b, ssem_b)

            for cp in ga:
                cp.wait()
            gb = start_gathers(ci_b, rows_b, gsem_b)
            process_bank(ci_a, rows_a, trans_a, ssem_a)
            for cp in gb:
                cp.wait()
            process_bank(ci_b, rows_b, trans_b, ssem_b)
            return carry

        lax.fori_loop(0, SUPER, body, 0)
        drain_stores(trans_a, ssem_a)
        drain_stores(trans_b, ssem_b)

    return gather_kernel


_gather = _embedding_gather_call()


def kernel(token_ids, weight):
    flat = token_ids.T.reshape(NUM_TOKENS // CHUNK, CHUNK).astype(jnp.int32)
    out = _gather(flat, weight)
    return out.reshape(NUM_T, DIM, NUM_B).transpose(2, 0, 1)
